# Initial kernel scaffold; baseline (speedup 1.0000x reference)
import jax
import jax.numpy as jnp
from jax.experimental import pallas as pl


def kernel(location, table, W, b):
    raise NotImplementedError("write your pallas kernel here")


# speedup vs baseline: 2.4330x; 2.4330x over previous
import functools

import jax
import jax.numpy as jnp
from jax import lax
from jax.experimental import pallas as pl
from jax.experimental.pallas import tpu as pltpu
from jax.experimental.pallas import tpu_sc as plsc

_N_LOCATIONS = 16384
_DIM = 64
_SIDE_BITS = 7
_MAX_DEPTH = 7
_BATCH = 4096
_N_LEAVES = 4 ** _MAX_DEPTH


def _expand_kernel(table_ref, W_ref, b_ref, out_ref):
    s = table_ref[0:1, :]
    for d in range(_MAX_DEPTH - 1):
        y = jnp.dot(s, W_ref[d], preferred_element_type=jnp.float32)
        y = y + b_ref[d][None, :]
        s = jnp.concatenate(
            [y[:, _DIM * c:_DIM * (c + 1)] for c in range(4)], axis=0)
    y = jnp.dot(s, W_ref[_MAX_DEPTH - 1], preferred_element_type=jnp.float32)
    out_ref[:, :] = y + b_ref[_MAX_DEPTH - 1][None, :]


def _expand_states(table, W, b):
    return pl.pallas_call(
        _expand_kernel,
        out_shape=jax.ShapeDtypeStruct((_N_LEAVES // 4, 4 * _DIM), jnp.float32),
    )(table, W, b)


_SC_INFO = plsc.get_sparse_core_info()
_NC = _SC_INFO.num_cores
_NW = _NC * _SC_INFO.num_subcores
_B_PER_W = _BATCH // _NW
_LANES = 16


def _leaf_row(loc):
    x = loc & (2 ** _SIDE_BITS - 1)
    y = loc >> _SIDE_BITS
    p = 2 * (y & 1) + (x & 1)
    for u in range(1, _SIDE_BITS):
        p = p + ((2 * ((y >> u) & 1) + ((x >> u) & 1)) << (2 * (_SIDE_BITS - u)))
    return p


@functools.partial(
    pl.kernel,
    mesh=plsc.VectorSubcoreMesh(core_axis_name="c", subcore_axis_name="s"),
    out_type=jax.ShapeDtypeStruct((_BATCH, _DIM), jnp.float32),
    scratch_types=[
        pltpu.VMEM((_B_PER_W,), jnp.int32),
        pltpu.VMEM((_B_PER_W, _DIM), jnp.float32),
        pltpu.SemaphoreType.DMA,
    ],
    compiler_params=pltpu.CompilerParams(use_tc_tiling_on_sc=False),
)
def _sc_gather(loc_hbm, leaf_hbm, out_hbm, idx_v, rows_v, sem):
    wid = lax.axis_index("s") * _NC + lax.axis_index("c")
    base = wid * _B_PER_W
    pltpu.sync_copy(loc_hbm.at[pl.ds(base, _B_PER_W)], idx_v)
    for i in range(_B_PER_W // _LANES):
        loc = idx_v[pl.ds(i * _LANES, _LANES)]
        idx_v[pl.ds(i * _LANES, _LANES)] = _leaf_row(loc)
    pltpu.async_copy(leaf_hbm.at[idx_v], rows_v, sem).wait()
    pltpu.sync_copy(rows_v, out_hbm.at[pl.ds(base, _B_PER_W)])


def kernel(location, table, W, b):
    leaf = _expand_states(table, W, b).reshape(_N_LEAVES, _DIM)
    return _sc_gather(location, leaf)


# speedup vs baseline: 2.4451x; 1.0050x over previous
import functools

import jax
import jax.numpy as jnp
from jax import lax
from jax.experimental import pallas as pl
from jax.experimental.pallas import tpu as pltpu
from jax.experimental.pallas import tpu_sc as plsc

_N_LOCATIONS = 16384
_DIM = 64
_SIDE_BITS = 7
_MAX_DEPTH = 7
_BATCH = 4096
_N_LEAVES = 4 ** _MAX_DEPTH


def _leaf_row(loc):
    x = loc & (2 ** _SIDE_BITS - 1)
    y = loc >> _SIDE_BITS
    p = 2 * (y & 1) + (x & 1)
    for u in range(1, _SIDE_BITS):
        p = p + ((2 * ((y >> u) & 1) + ((x >> u) & 1)) << (2 * (_SIDE_BITS - u)))
    return p


def _expand_kernel(loc_ref, table_ref, W_ref, b_ref, out_ref, idx_ref):
    idx_ref[:, :] = _leaf_row(loc_ref[:, :])
    s = table_ref[0:1, :]
    for d in range(_MAX_DEPTH - 1):
        y = jnp.dot(s, W_ref[d], preferred_element_type=jnp.float32)
        y = y + b_ref[d][None, :]
        s = jnp.concatenate(
            [y[:, _DIM * c:_DIM * (c + 1)] for c in range(4)], axis=0)
    y = jnp.dot(s, W_ref[_MAX_DEPTH - 1], preferred_element_type=jnp.float32)
    out_ref[:, :] = y + b_ref[_MAX_DEPTH - 1][None, :]


def _expand_states(location, table, W, b):
    return pl.pallas_call(
        _expand_kernel,
        out_shape=(
            jax.ShapeDtypeStruct((_N_LEAVES // 4, 4 * _DIM), jnp.float32),
            jax.ShapeDtypeStruct((_BATCH // 128, 128), jnp.int32),
        ),
    )(location.reshape(_BATCH // 128, 128), table, W, b)


_SC_INFO = plsc.get_sparse_core_info()
_NC = _SC_INFO.num_cores
_NW = _NC * _SC_INFO.num_subcores
_B_PER_W = _BATCH // _NW


@functools.partial(
    pl.kernel,
    mesh=plsc.VectorSubcoreMesh(core_axis_name="c", subcore_axis_name="s"),
    out_type=jax.ShapeDtypeStruct((_BATCH, _DIM), jnp.float32),
    scratch_types=[
        pltpu.VMEM((_B_PER_W,), jnp.int32),
        pltpu.VMEM((_B_PER_W, _DIM), jnp.float32),
        pltpu.SemaphoreType.DMA,
    ],
    compiler_params=pltpu.CompilerParams(use_tc_tiling_on_sc=False),
)
def _sc_gather(idx_hbm, leaf_hbm, out_hbm, idx_v, rows_v, sem):
    wid = lax.axis_index("s") * _NC + lax.axis_index("c")
    base = wid * _B_PER_W
    pltpu.sync_copy(idx_hbm.at[pl.ds(base, _B_PER_W)], idx_v)
    pltpu.async_copy(leaf_hbm.at[idx_v], rows_v, sem).wait()
    pltpu.sync_copy(rows_v, out_hbm.at[pl.ds(base, _B_PER_W)])


def kernel(location, table, W, b):
    leaf4, idx = _expand_states(location, table, W, b)
    leaf = leaf4.reshape(_N_LEAVES, _DIM)
    return _sc_gather(idx.reshape(_BATCH), leaf)


# speedup vs baseline: 2.4481x; 1.0012x over previous
import functools

import jax
import jax.numpy as jnp
from jax import lax
from jax.experimental import pallas as pl
from jax.experimental.pallas import tpu as pltpu
from jax.experimental.pallas import tpu_sc as plsc

_N_LOCATIONS = 16384
_DIM = 64
_SIDE_BITS = 7
_MAX_DEPTH = 7
_BATCH = 4096
_N_LEAVES = 4 ** _MAX_DEPTH


def _leaf_row(loc):
    x = loc & (2 ** _SIDE_BITS - 1)
    y = loc >> _SIDE_BITS
    p = 2 * (y & 1) + (x & 1)
    for u in range(1, _SIDE_BITS):
        p = p + ((2 * ((y >> u) & 1) + ((x >> u) & 1)) << (2 * (_SIDE_BITS - u)))
    return p


def _expand_kernel(loc_ref, table_ref, W_ref, b_ref, out_ref, idx_ref):
    idx_ref[:, :] = _leaf_row(loc_ref[:, :])
    s = table_ref[0:1, :]
    for d in range(_MAX_DEPTH - 1):
        y = jnp.dot(s, W_ref[d], preferred_element_type=jnp.float32)
        y = y + b_ref[d][None, :]
        s = jnp.concatenate(
            [y[:, _DIM * c:_DIM * (c + 1)] for c in range(4)], axis=0)
    y = jnp.dot(s, W_ref[_MAX_DEPTH - 1], preferred_element_type=jnp.float32)
    out_ref[:, :] = y + b_ref[_MAX_DEPTH - 1][None, :]


def _expand_states(location, table, W, b):
    return pl.pallas_call(
        _expand_kernel,
        out_shape=(
            jax.ShapeDtypeStruct((_N_LEAVES // 4, 4 * _DIM), jnp.float32),
            jax.ShapeDtypeStruct((_BATCH // 128, 128), jnp.int32),
        ),
    )(location.reshape(_BATCH // 128, 128), table, W, b)


_SC_INFO = plsc.get_sparse_core_info()
_NC = _SC_INFO.num_cores
_NW = _NC * _SC_INFO.num_subcores
_B_PER_W = _BATCH // _NW


@functools.partial(
    pl.kernel,
    mesh=plsc.VectorSubcoreMesh(core_axis_name="c", subcore_axis_name="s"),
    out_type=jax.ShapeDtypeStruct((_BATCH, _DIM), jnp.float32),
    scratch_types=[
        pltpu.VMEM((_B_PER_W,), jnp.int32),
        pltpu.VMEM((_B_PER_W, _DIM), jnp.float32),
        pltpu.SemaphoreType.DMA,
    ],
    compiler_params=pltpu.CompilerParams(
        use_tc_tiling_on_sc=False,
        disable_bounds_checks=True,
        disable_semaphore_checks=True,
        skip_device_barrier=True,
    ),
)
def _sc_gather(idx_hbm, leaf_hbm, out_hbm, idx_v, rows_v, sem):
    wid = lax.axis_index("s") * _NC + lax.axis_index("c")
    base = wid * _B_PER_W
    pltpu.sync_copy(idx_hbm.at[pl.ds(base, _B_PER_W)], idx_v)
    pltpu.async_copy(leaf_hbm.at[idx_v], rows_v, sem).wait()
    pltpu.sync_copy(rows_v, out_hbm.at[pl.ds(base, _B_PER_W)])


def kernel(location, table, W, b):
    leaf4, idx = _expand_states(location, table, W, b)
    leaf = leaf4.reshape(_N_LEAVES, _DIM)
    return _sc_gather(idx.reshape(_BATCH), leaf)


# speedup vs baseline: 2.7352x; 1.1173x over previous
import functools

import jax
import jax.numpy as jnp
from jax import lax
from jax.experimental import pallas as pl
from jax.experimental.pallas import tpu as pltpu
from jax.experimental.pallas import tpu_sc as plsc

_N_LOCATIONS = 16384
_DIM = 64
_SIDE_BITS = 7
_MAX_DEPTH = 7
_BATCH = 4096
_N_LEAVES = 4 ** _MAX_DEPTH


def _leaf_row(loc):
    x = loc & (2 ** _SIDE_BITS - 1)
    y = loc >> _SIDE_BITS
    p = (2 * (y & 1) + (x & 1)) << (2 * (_SIDE_BITS - 1))
    for u in range(1, _SIDE_BITS):
        p = p + ((2 * ((y >> u) & 1) + ((x >> u) & 1)) << (2 * (_SIDE_BITS - 1 - u)))
    return p


def _expand_kernel(loc_ref, table_ref, W_ref, b_ref, out_ref, idx_ref):
    idx_ref[:, :] = _leaf_row(loc_ref[:, :])
    s = table_ref[0:1, :]
    for d in range(_MAX_DEPTH - 1):
        y = jnp.dot(s, W_ref[d], preferred_element_type=jnp.float32)
        y = y + b_ref[d][None, :]
        s = jnp.concatenate(
            [y[:, _DIM * c:_DIM * (c + 1)] for c in range(4)], axis=0)
    y = jnp.dot(s, W_ref[_MAX_DEPTH - 1], preferred_element_type=jnp.float32)
    y = y + b_ref[_MAX_DEPTH - 1][None, :]
    for c in range(4):
        out_ref[c * (_N_LEAVES // 4):(c + 1) * (_N_LEAVES // 4), 0:_DIM] = (
            y[:, _DIM * c:_DIM * (c + 1)])


def _expand_states(location, table, W, b):
    return pl.pallas_call(
        _expand_kernel,
        out_shape=(
            jax.ShapeDtypeStruct((_N_LEAVES, 2 * _DIM), jnp.float32),
            jax.ShapeDtypeStruct((_BATCH // 128, 128), jnp.int32),
        ),
    )(location.reshape(_BATCH // 128, 128), table, W, b)


_SC_INFO = plsc.get_sparse_core_info()
_NC = _SC_INFO.num_cores
_NW = _NC * _SC_INFO.num_subcores
_B_PER_W = _BATCH // _NW


@functools.partial(
    pl.kernel,
    mesh=plsc.VectorSubcoreMesh(core_axis_name="c", subcore_axis_name="s"),
    out_type=jax.ShapeDtypeStruct((_BATCH, 2 * _DIM), jnp.float32),
    scratch_types=[
        pltpu.VMEM((_B_PER_W,), jnp.int32),
        pltpu.VMEM((_B_PER_W, 2 * _DIM), jnp.float32),
        pltpu.SemaphoreType.DMA,
    ],
)
def _sc_gather(idx_hbm, leaf_hbm, out_hbm, idx_v, rows_v, sem):
    wid = lax.axis_index("s") * _NC + lax.axis_index("c")
    base = wid * _B_PER_W
    pltpu.sync_copy(idx_hbm.at[pl.ds(base, _B_PER_W)], idx_v)
    pltpu.async_copy(leaf_hbm.at[idx_v], rows_v, sem).wait()
    pltpu.sync_copy(rows_v, out_hbm.at[pl.ds(base, _B_PER_W)])


def kernel(location, table, W, b):
    leaf, idx = _expand_states(location, table, W, b)
    padded = _sc_gather(idx.reshape(_BATCH), leaf)
    return padded[:, :_DIM]
